# Initial kernel scaffold; baseline (speedup 1.0000x reference)
import jax
import jax.numpy as jnp
from jax.experimental import pallas as pl


def kernel(smpl_v, object_v, smpl_contact_maps, object_contact_maps):
    raise NotImplementedError("write your pallas kernel here")


# speedup vs baseline: 1.0453x; 1.0453x over previous
import functools

import jax
import jax.numpy as jnp
from jax.experimental import pallas as pl
from jax.experimental.pallas import tpu as pltpu

B = 16
P1 = 6890
P2 = 4096
TI = 768
P1PAD = 6912
NI = P1PAD // TI
PAD_COORD = 1.0e6


def _body(x_ref, y_ref, scm_ref, ocm_ref, out_ref, colmin_ref):
    ni = pl.program_id(1)

    x = x_ref[0]
    y = y_ref[0]
    x2 = jnp.sum(x * x, axis=1, keepdims=True)
    y2 = jnp.sum(y * y, axis=1)[None, :]
    xy = jax.lax.dot_general(
        x, y, (((1,), (1,)), ((), ())), preferred_element_type=jnp.float32
    )
    d = x2 + y2 - 2.0 * xy

    @pl.when(ni == 0)
    def _init():
        colmin_ref[...] = jnp.full_like(colmin_ref, jnp.inf)
        out_ref[0, 0, 0] = 0.0
        out_ref[0, 0, 1] = 0.0
        out_ref[0, 0, 2] = 0.0
        out_ref[0, 0, 3] = 0.0

    scm = scm_ref[0, 0]
    rowmin = jnp.maximum(jnp.min(d, axis=1), 0.0)
    out_ref[0, 0, 0] += jnp.sum(scm * rowmin)
    out_ref[0, 0, 1] += jnp.sum(scm)

    colmin_ref[...] = jnp.minimum(colmin_ref[...], jnp.min(d, axis=0, keepdims=True))

    @pl.when(ni == NI - 1)
    def _fini():
        ocm = ocm_ref[0, 0]
        colmin = jnp.maximum(colmin_ref[0], 0.0)
        out_ref[0, 0, 2] = jnp.sum(ocm * colmin)
        out_ref[0, 0, 3] = jnp.sum(ocm)


@jax.jit
def kernel(smpl_v, object_v, smpl_contact_maps, object_contact_maps):
    xpad = jnp.pad(smpl_v, ((0, 0), (0, P1PAD - P1), (0, 0)),
                   constant_values=PAD_COORD)
    scm = jnp.pad(smpl_contact_maps[..., 0], ((0, 0), (0, P1PAD - P1)))
    scm = scm[:, None, :]
    ocm = object_contact_maps[..., 0][:, None, :]

    parts = pl.pallas_call(
        _body,
        grid=(B, NI),
        in_specs=[
            pl.BlockSpec((1, TI, 3), lambda b, i: (b, i, 0)),
            pl.BlockSpec((1, P2, 3), lambda b, i: (b, 0, 0)),
            pl.BlockSpec((1, 1, TI), lambda b, i: (b, 0, i)),
            pl.BlockSpec((1, 1, P2), lambda b, i: (b, 0, 0)),
        ],
        out_specs=pl.BlockSpec((1, 1, 4), lambda b, i: (b, 0, 0),
                               memory_space=pltpu.SMEM),
        out_shape=jax.ShapeDtypeStruct((B, 1, 4), jnp.float32),
        scratch_shapes=[pltpu.VMEM((1, P2), jnp.float32)],
    )(xpad, object_v, scm, ocm)

    parts = parts[:, 0]
    loss = jnp.sum(parts[:, 0] / (parts[:, 1] + 1e-6)
                   + parts[:, 2] / (parts[:, 3] + 1e-6)) / B
    return loss
